# baseline (speedup 1.0000x reference)
import functools

import jax
import jax.numpy as jnp
from jax import lax
from jax.experimental import pallas as pl
from jax.experimental.pallas import tpu as pltpu
from jax.experimental.pallas import tpu_sc as plsc

_NUM_USER = 1000000
_DIM = 32
_BATCH = 16384

_INFO = plsc.get_sparse_core_info()
_NC = _INFO.num_cores
_NS = _INFO.num_subcores
_NW = _NC * _NS
_B_PER_W = _BATCH // _NW


@functools.partial(
    pl.kernel,
    mesh=plsc.VectorSubcoreMesh(core_axis_name="c", subcore_axis_name="s"),
    out_type=jax.ShapeDtypeStruct((_BATCH, _DIM), jnp.float32),
    scratch_types=[
        pltpu.VMEM((_B_PER_W,), jnp.int32),
        pltpu.VMEM((_B_PER_W, _DIM), jnp.float32),
        pltpu.SemaphoreType.DMA((16,)),
    ],
)
def _sc_gather(table_hbm, idx_hbm, out_hbm, idx_v, rows_v, sem):
    wid = lax.axis_index("s") * _NC + lax.axis_index("c")
    base = wid * _B_PER_W
    pltpu.sync_copy(idx_hbm.at[pl.ds(base, _B_PER_W)], idx_v)

    @plsc.parallel_loop(0, _B_PER_W // 16, 1, unroll=2)
    def fire(g):
        v = idx_v[pl.ds(g * 16, 16)]
        for j in range(16):
            row = v[j]
            pltpu.async_copy(
                table_hbm.at[pl.ds(row, 1), :],
                rows_v.at[pl.ds(g * 16 + j, 1), :],
                sem.at[j],
            )

    def drain(i, carry):
        for j in range(16):
            pltpu.make_async_copy(
                table_hbm.at[pl.ds(0, 1), :], rows_v.at[pl.ds(0, 1), :], sem.at[j]
            ).wait()
        return carry

    lax.fori_loop(0, _B_PER_W // 16, drain, 0)
    pltpu.sync_copy(rows_v, out_hbm.at[pl.ds(base, _B_PER_W)])


def kernel(user_fea, embedding_userId):
    idx = user_fea[:, 0].astype(jnp.int32)
    return _sc_gather(embedding_userId, idx)


# speedup vs baseline: 1.0811x; 1.0811x over previous
import functools

import jax
import jax.numpy as jnp
from jax import lax
from jax.experimental import pallas as pl
from jax.experimental.pallas import tpu as pltpu
from jax.experimental.pallas import tpu_sc as plsc

_NUM_USER = 1000000
_DIM = 32
_BATCH = 16384

_INFO = plsc.get_sparse_core_info()
_NC = _INFO.num_cores
_NS = _INFO.num_subcores
_NW = _NC * _NS
_B_PER_W = _BATCH // _NW


@functools.partial(
    pl.kernel,
    mesh=plsc.VectorSubcoreMesh(core_axis_name="c", subcore_axis_name="s"),
    out_type=jax.ShapeDtypeStruct((_BATCH, _DIM), jnp.float32),
    scratch_types=[
        pltpu.VMEM((_B_PER_W,), jnp.int32),
        pltpu.VMEM((_B_PER_W, _DIM), jnp.float32),
        pltpu.SemaphoreType.DMA,
    ],
    compiler_params=pltpu.CompilerParams(use_tc_tiling_on_sc=True),
)
def _sc_gather(table_hbm, idx_hbm, out_hbm, idx_v, rows_v, sem):
    wid = lax.axis_index("s") * _NC + lax.axis_index("c")
    base = wid * _B_PER_W
    pltpu.sync_copy(idx_hbm.at[pl.ds(base, _B_PER_W)], idx_v)

    @plsc.parallel_loop(0, _B_PER_W // 16, 1, unroll=2)
    def fire(g):
        v = idx_v[pl.ds(g * 16, 16)]
        for j in range(16):
            row = v[j]
            pltpu.async_copy(
                table_hbm.at[pl.ds(row, 1), :],
                rows_v.at[pl.ds(g * 16 + j, 1), :],
                sem,
            )

    def drain(i, carry):
        pltpu.make_async_copy(
            table_hbm.at[pl.ds(0, 1), :], rows_v.at[pl.ds(0, 1), :], sem
        ).wait()
        return carry

    lax.fori_loop(0, _B_PER_W, drain, 0)
    pltpu.sync_copy(rows_v, out_hbm.at[pl.ds(base, _B_PER_W)])


def kernel(user_fea, embedding_userId):
    idx = user_fea[:, 0].astype(jnp.int32)
    return _sc_gather(embedding_userId, idx)


# speedup vs baseline: 1.1223x; 1.0381x over previous
import functools

import jax
import jax.numpy as jnp
from jax import lax
from jax.experimental import pallas as pl
from jax.experimental.pallas import tpu as pltpu
from jax.experimental.pallas import tpu_sc as plsc

_NUM_USER = 1000000
_DIM = 32
_BATCH = 16384

_INFO = plsc.get_sparse_core_info()
_NC = _INFO.num_cores
_NS = _INFO.num_subcores
_NW = _NC * _NS
_K = 1024
_NRANGE = 977
_ROUNDS = 31
_CAP = 64
_NVEC = _BATCH // 16


@functools.partial(
    pl.kernel,
    mesh=plsc.VectorSubcoreMesh(core_axis_name="c", subcore_axis_name="s"),
    out_type=jax.ShapeDtypeStruct((_BATCH, _DIM), jnp.float32),
    scratch_types=[
        pltpu.VMEM((_BATCH,), jnp.int32),
        pltpu.VMEM((_DIM, _K), jnp.float32),
        pltpu.VMEM((_ROUNDS * _CAP,), jnp.int32),
        pltpu.VMEM((_ROUNDS * _CAP,), jnp.int32),
        pltpu.VMEM((32,), jnp.int32),
        pltpu.VMEM((32,), jnp.int32),
        pltpu.VMEM((_CAP, _DIM), jnp.float32),
        pltpu.SemaphoreType.DMA,
    ],
    compiler_params=pltpu.CompilerParams(use_tc_tiling_on_sc=True, needs_layout_passes=False),
)
def _sc_gather(tableT_hbm, tail_hbm, idx_hbm, out_hbm,
               idx_v, chunk, bk_u, bk_p, cnts, ovf, stage, wsem):
    w = lax.axis_index("s") * _NC + lax.axis_index("c")
    lanes = lax.iota(jnp.int32, 16)
    zeros16 = jnp.zeros((16,), jnp.int32)
    lane0 = lanes == 0

    pltpu.sync_copy(idx_hbm, idx_v)
    cnts[pl.ds(0, 16)] = zeros16
    cnts[pl.ds(16, 16)] = zeros16
    ovf[pl.ds(0, 16)] = zeros16
    ovf[pl.ds(16, 16)] = zeros16

    def scan(g, carry):
        v = idx_v[pl.ds(g * 16, 16)]
        rid = lax.shift_right_logical(v, 10)
        mine = (rid & 31) == w
        np_ = plsc.all_reduce_population_count(mine)[0]

        @pl.when(np_ > 0)
        def _():
            mine_i = jnp.where(mine, 1, 0)
            rnd = lax.shift_right_logical(rid, 5)
            for j in range(16):
                @pl.when(mine_i[j] > 0)
                def _():
                    rj = rnd[j]
                    cv = plsc.load_gather(cnts, [zeros16 + rj])
                    slot = cv[0]
                    okm = lane0 & jnp.broadcast_to(slot < _CAP, (16,))
                    ovm = lane0 & jnp.broadcast_to(slot >= _CAP, (16,))
                    pos = rj * _CAP + jnp.minimum(slot, _CAP - 1)
                    plsc.store_scatter(bk_u, [zeros16 + pos], zeros16 + v[j],
                                       mask=okm)
                    plsc.store_scatter(bk_p, [zeros16 + pos],
                                       zeros16 + (g * 16 + j), mask=okm)
                    plsc.store_scatter(cnts, [zeros16 + rj],
                                       zeros16 + (slot + 1), mask=okm)
                    plsc.store_scatter(ovf, [zeros16 + rj],
                                       zeros16 + 1, mask=ovm)
        return carry

    lax.fori_loop(0, _NVEC, scan, 0)

    dlo = lanes
    dhi = lanes + 16

    def extract_hit(u_scalar, p_scalar, slot):
        ul = zeros16 + (u_scalar & (_K - 1))
        lo = plsc.load_gather(chunk, [dlo, ul])
        hi = plsc.load_gather(chunk, [dhi, ul])
        stage[slot, pl.ds(0, 16)] = lo
        stage[slot, pl.ds(16, 16)] = hi
        pltpu.async_copy(
            stage.at[pl.ds(slot, 1), :],
            out_hbm.at[pl.ds(p_scalar, 1), :],
            wsem,
        )

    def drain_n(n):
        def d(i, c):
            pltpu.make_async_copy(
                stage.at[pl.ds(0, 1), :], out_hbm.at[pl.ds(0, 1), :], wsem
            ).wait()
            return c
        lax.fori_loop(0, n, d, 0)

    def rnd(r, carry):
        rid = r * 32 + w
        c0 = rid * _K

        @pl.when(rid < _NRANGE - 1)
        def _():
            pltpu.sync_copy(tableT_hbm.at[:, pl.ds(c0, _K)], chunk)

        @pl.when(rid == _NRANGE - 1)
        def _():
            pltpu.sync_copy(
                tableT_hbm.at[:, pl.ds((_NRANGE - 1) * _K, 512)],
                chunk.at[:, pl.ds(0, 512)],
            )
            pltpu.sync_copy(tail_hbm, chunk.at[:, pl.ds(512, 128)])

        nv = plsc.load_gather(cnts, [zeros16 + r])
        n = nv[0]
        ov = plsc.load_gather(ovf, [zeros16 + r])[0]

        @pl.when(ov == 0)
        def _():
            def grp(k, hc):
                uv = bk_u[pl.ds(r * _CAP + k * 16, 16)]
                pv = bk_p[pl.ds(r * _CAP + k * 16, 16)]
                for j in range(16):
                    @pl.when((k * 16 + j) < n)
                    def _():
                        extract_hit(uv[j], pv[j], k * 16 + j)
                return hc
            lax.fori_loop(0, (_CAP + 15) // 16, grp, 0)
            drain_n(n)

        @pl.when(ov != 0)
        def _():
            def rescan(g, hc):
                v = idx_v[pl.ds(g * 16, 16)]
                hit = lax.shift_right_logical(v, 10) == rid
                np_ = plsc.all_reduce_population_count(hit)[0]

                @pl.when(np_ > 0)
                def _():
                    hit_i = jnp.where(hit, 1, 0)
                    for j in range(16):
                        @pl.when(hit_i[j] > 0)
                        def _():
                            extract_hit(v[j], g * 16 + j, j)
                    drain_n(np_)
                return hc
            lax.fori_loop(0, _NVEC, rescan, 0)
        return carry

    lax.fori_loop(0, _ROUNDS, rnd, 0)


def kernel(user_fea, embedding_userId):
    idx = user_fea[:, 0].astype(jnp.int32)
    tail = jnp.pad(embedding_userId[999936:].T, ((0, 0), (0, 64)))
    return _sc_gather(embedding_userId.T, tail, idx)


# speedup vs baseline: 2.2739x; 2.0261x over previous
import functools

import jax
import jax.numpy as jnp
from jax import lax
from jax.experimental import pallas as pl
from jax.experimental.pallas import tpu as pltpu
from jax.experimental.pallas import tpu_sc as plsc

_NUM_USER = 1000000
_DIM = 32
_BATCH = 16384

_INFO = plsc.get_sparse_core_info()
_NC = _INFO.num_cores
_NS = _INFO.num_subcores
_NW = _NC * _NS
_K = 1024
_NRANGE = 977
_ROUNDS = 31
_CAP = 64
_NVEC = _BATCH // 16


@functools.partial(
    pl.kernel,
    mesh=plsc.VectorSubcoreMesh(core_axis_name="c", subcore_axis_name="s"),
    out_type=jax.ShapeDtypeStruct((_BATCH, _DIM), jnp.float32),
    scratch_types=[
        pltpu.VMEM((_BATCH,), jnp.int32),
        pltpu.VMEM((_DIM, _K), jnp.float32),
        pltpu.VMEM((_ROUNDS * _CAP,), jnp.int32),
        pltpu.VMEM((_ROUNDS * _CAP,), jnp.int32),
        pltpu.VMEM((32,), jnp.int32),
        pltpu.VMEM((32,), jnp.int32),
        pltpu.VMEM((_CAP, _DIM), jnp.float32),
        pltpu.SemaphoreType.DMA,
    ],
    compiler_params=pltpu.CompilerParams(use_tc_tiling_on_sc=True, needs_layout_passes=False),
)
def _sc_gather(tableT_hbm, tail_hbm, idx_hbm, out_hbm,
               idx_v, chunk, bk_u, bk_p, cnts, ovf, stage, wsem):
    w = lax.axis_index("s") * _NC + lax.axis_index("c")
    lanes = lax.iota(jnp.int32, 16)
    zeros16 = jnp.zeros((16,), jnp.int32)
    lane0 = lanes == 0

    pltpu.sync_copy(idx_hbm, idx_v)
    cnts[pl.ds(0, 16)] = zeros16
    cnts[pl.ds(16, 16)] = zeros16
    ovf[pl.ds(0, 16)] = zeros16
    ovf[pl.ds(16, 16)] = zeros16

    def scan(g, carry):
        v = idx_v[pl.ds(g * 16, 16)]
        rid = lax.shift_right_logical(v, 10)
        mine = (rid & 31) == w

        def has_hits(st):
            return plsc.all_reduce_population_count(st[0])[0] > 0

        def take_hit(st):
            m, _ = st
            j = plsc.all_reduce_ffs(m)[0]
            jb = zeros16 + j
            ub = jnp.where(lanes == jb, v, 0)
            ub = zeros16 + plsc.cumsum(ub)[15]
            rj = lax.shift_right_logical(ub, 15)
            cv = plsc.load_gather(cnts, [rj])
            ok = cv < _CAP
            okm = lane0 & ok
            ovm = lane0 & jnp.logical_not(ok)
            pos = rj * _CAP + jnp.minimum(cv, _CAP - 1)
            plsc.store_scatter(bk_u, [pos], ub, mask=okm)
            plsc.store_scatter(bk_p, [pos], zeros16 + (g * 16 + j), mask=okm)
            plsc.store_scatter(cnts, [rj], cv + 1, mask=okm)
            plsc.store_scatter(ovf, [rj], zeros16 + 1, mask=ovm)
            return (m & (lanes != jb), 0)

        lax.while_loop(has_hits, take_hit, (mine, 0))
        return carry

    lax.fori_loop(0, _NVEC, scan, 0)

    dlo = lanes
    dhi = lanes + 16

    def extract_hit(u_scalar, p_scalar, slot):
        ul = zeros16 + (u_scalar & (_K - 1))
        lo = plsc.load_gather(chunk, [dlo, ul])
        hi = plsc.load_gather(chunk, [dhi, ul])
        stage[slot, pl.ds(0, 16)] = lo
        stage[slot, pl.ds(16, 16)] = hi
        pltpu.async_copy(
            stage.at[pl.ds(slot, 1), :],
            out_hbm.at[pl.ds(p_scalar, 1), :],
            wsem,
        )

    def drain_n(n):
        def d(i, c):
            pltpu.make_async_copy(
                stage.at[pl.ds(0, 1), :], out_hbm.at[pl.ds(0, 1), :], wsem
            ).wait()
            return c
        lax.fori_loop(0, n, d, 0)

    def rnd(r, carry):
        rid = r * 32 + w
        c0 = rid * _K

        @pl.when(rid < _NRANGE - 1)
        def _():
            pltpu.sync_copy(tableT_hbm.at[:, pl.ds(c0, _K)], chunk)

        @pl.when(rid == _NRANGE - 1)
        def _():
            pltpu.sync_copy(
                tableT_hbm.at[:, pl.ds((_NRANGE - 1) * _K, 512)],
                chunk.at[:, pl.ds(0, 512)],
            )
            pltpu.sync_copy(tail_hbm, chunk.at[:, pl.ds(512, 128)])

        nv = plsc.load_gather(cnts, [zeros16 + r])
        n = nv[0]
        ov = plsc.load_gather(ovf, [zeros16 + r])[0]

        @pl.when(ov == 0)
        def _():
            def hit(h, hc):
                hb = zeros16 + (r * _CAP + h)
                ub = plsc.load_gather(bk_u, [hb])
                pb = plsc.load_gather(bk_p, [hb])
                extract_hit(ub[0], pb[0], h)
                return hc
            lax.fori_loop(0, n, hit, 0)
            drain_n(n)

        @pl.when(ov != 0)
        def _():
            def rescan(g, hc):
                v = idx_v[pl.ds(g * 16, 16)]
                hit = lax.shift_right_logical(v, 10) == rid
                np_ = plsc.all_reduce_population_count(hit)[0]

                @pl.when(np_ > 0)
                def _():
                    hit_i = jnp.where(hit, 1, 0)
                    for j in range(16):
                        @pl.when(hit_i[j] > 0)
                        def _():
                            extract_hit(v[j], g * 16 + j, j)
                    drain_n(np_)
                return hc
            lax.fori_loop(0, _NVEC, rescan, 0)
        return carry

    lax.fori_loop(0, _ROUNDS, rnd, 0)


def kernel(user_fea, embedding_userId):
    idx = user_fea[:, 0].astype(jnp.int32)
    tail = jnp.pad(embedding_userId[999936:].T, ((0, 0), (0, 64)))
    return _sc_gather(embedding_userId.T, tail, idx)


# speedup vs baseline: 2.7416x; 1.2057x over previous
import functools

import jax
import jax.numpy as jnp
from jax import lax
from jax.experimental import pallas as pl
from jax.experimental.pallas import tpu as pltpu
from jax.experimental.pallas import tpu_sc as plsc

_NUM_USER = 1000000
_DIM = 32
_BATCH = 16384

_INFO = plsc.get_sparse_core_info()
_NC = _INFO.num_cores
_NS = _INFO.num_subcores
_NW = _NC * _NS
_K = 1024
_NRANGE = 977
_ROUNDS = 31
_CAP = 64
_NVEC = _BATCH // 16


@functools.partial(
    pl.kernel,
    mesh=plsc.VectorSubcoreMesh(core_axis_name="c", subcore_axis_name="s"),
    out_type=jax.ShapeDtypeStruct((_BATCH, _DIM), jnp.float32),
    scratch_types=[
        pltpu.VMEM((_BATCH,), jnp.int32),
        pltpu.VMEM((_DIM, _K), jnp.float32),
        pltpu.VMEM((_DIM, _K), jnp.float32),
        pltpu.VMEM((_ROUNDS * _CAP,), jnp.int32),
        pltpu.VMEM((_ROUNDS * _CAP,), jnp.int32),
        pltpu.VMEM((32,), jnp.int32),
        pltpu.VMEM((32,), jnp.int32),
        pltpu.VMEM((_CAP, _DIM), jnp.float32),
        pltpu.SemaphoreType.DMA,
        pltpu.SemaphoreType.DMA((2,)),
    ],
    compiler_params=pltpu.CompilerParams(use_tc_tiling_on_sc=True, needs_layout_passes=False),
)
def _sc_gather(tableT_hbm, tail_hbm, idx_hbm, out_hbm,
               idx_v, chunk0, chunk1, bk_u, bk_p, cnts, ovf, stage, wsem,
               csem):
    w = lax.axis_index("s") * _NC + lax.axis_index("c")
    lanes = lax.iota(jnp.int32, 16)
    zeros16 = jnp.zeros((16,), jnp.int32)
    lane0 = lanes == 0

    pltpu.sync_copy(idx_hbm, idx_v)
    cnts[pl.ds(0, 16)] = zeros16
    cnts[pl.ds(16, 16)] = zeros16
    ovf[pl.ds(0, 16)] = zeros16
    ovf[pl.ds(16, 16)] = zeros16

    def scan(g, carry):
        v = idx_v[pl.ds(g * 16, 16)]
        rid = lax.shift_right_logical(v, 10)
        mine = (rid & 31) == w

        def has_hits(st):
            return plsc.all_reduce_population_count(st[0])[0] > 0

        def take_hit(st):
            m, _ = st
            j = plsc.all_reduce_ffs(m)[0]
            jb = zeros16 + j
            ub = jnp.where(lanes == jb, v, 0)
            ub = zeros16 + plsc.cumsum(ub)[15]
            rj = lax.shift_right_logical(ub, 15)
            cv = plsc.load_gather(cnts, [rj])
            ok = cv < _CAP
            okm = lane0 & ok
            ovm = lane0 & jnp.logical_not(ok)
            pos = rj * _CAP + jnp.minimum(cv, _CAP - 1)
            plsc.store_scatter(bk_u, [pos], ub, mask=okm)
            plsc.store_scatter(bk_p, [pos], zeros16 + (g * 16 + j), mask=okm)
            plsc.store_scatter(cnts, [rj], cv + 1, mask=okm)
            plsc.store_scatter(ovf, [rj], zeros16 + 1, mask=ovm)
            return (m & (lanes != jb), 0)

        lax.while_loop(has_hits, take_hit, (mine, 0))
        return carry

    lax.fori_loop(0, _NVEC, scan, 0)

    dlo = lanes
    dhi = lanes + 16

    def extract_hit(chunk, u_scalar, p_scalar, slot):
        ul = zeros16 + (u_scalar & (_K - 1))
        lo = plsc.load_gather(chunk, [dlo, ul])
        hi = plsc.load_gather(chunk, [dhi, ul])
        stage[slot, pl.ds(0, 16)] = lo
        stage[slot, pl.ds(16, 16)] = hi
        pltpu.async_copy(
            stage.at[pl.ds(slot, 1), :],
            out_hbm.at[pl.ds(p_scalar, 1), :],
            wsem,
        )

    def drain_n(n):
        def d(i, c):
            pltpu.make_async_copy(
                stage.at[pl.ds(0, 1), :], out_hbm.at[pl.ds(0, 1), :], wsem
            ).wait()
            return c
        lax.fori_loop(0, n, d, 0)

    def fire(r, chunk, p):
        rid = r * 32 + w

        @pl.when(rid < _NRANGE - 1)
        def _():
            pltpu.async_copy(
                tableT_hbm.at[:, pl.ds(rid * _K, _K)], chunk, csem.at[p]
            )

        @pl.when(rid == _NRANGE - 1)
        def _():
            pltpu.async_copy(
                tableT_hbm.at[:, pl.ds((_NRANGE - 1) * _K, 512)],
                chunk.at[:, pl.ds(0, 512)],
                csem.at[p],
            )
            pltpu.async_copy(tail_hbm, chunk.at[:, pl.ds(512, 128)], csem.at[p])

    def wait_chunk(r, chunk, p):
        rid = r * 32 + w

        @pl.when(rid < _NRANGE - 1)
        def _():
            pltpu.make_async_copy(
                tableT_hbm.at[:, pl.ds(0, _K)], chunk, csem.at[p]
            ).wait()

        @pl.when(rid == _NRANGE - 1)
        def _():
            pltpu.make_async_copy(
                tableT_hbm.at[:, pl.ds(0, 512)],
                chunk.at[:, pl.ds(0, 512)],
                csem.at[p],
            ).wait()
            pltpu.make_async_copy(
                tail_hbm, chunk.at[:, pl.ds(512, 128)], csem.at[p]
            ).wait()

    def serve(r, chunk):
        rid = r * 32 + w
        nv = plsc.load_gather(cnts, [zeros16 + r])
        n = nv[0]
        ov = plsc.load_gather(ovf, [zeros16 + r])[0]

        @pl.when(ov == 0)
        def _():
            def hit(h, hc):
                hb = zeros16 + (r * _CAP + h)
                ub = plsc.load_gather(bk_u, [hb])
                pb = plsc.load_gather(bk_p, [hb])
                extract_hit(chunk, ub[0], pb[0], h)
                return hc
            lax.fori_loop(0, n, hit, 0)
            drain_n(n)

        @pl.when(ov != 0)
        def _():
            def rescan(g, hc):
                v = idx_v[pl.ds(g * 16, 16)]
                hit = lax.shift_right_logical(v, 10) == rid
                np_ = plsc.all_reduce_population_count(hit)[0]

                @pl.when(np_ > 0)
                def _():
                    hit_i = jnp.where(hit, 1, 0)
                    for j in range(16):
                        @pl.when(hit_i[j] > 0)
                        def _():
                            extract_hit(chunk, v[j], g * 16 + j, j)
                    drain_n(np_)
                return hc
            lax.fori_loop(0, _NVEC, rescan, 0)

    fire(0, chunk0, 0)

    def rnd2(t, carry):
        a = t * 2
        fire(a + 1, chunk1, 1)
        wait_chunk(a, chunk0, 0)
        serve(a, chunk0)

        @pl.when(a + 2 < _ROUNDS)
        def _():
            fire(a + 2, chunk0, 0)
        wait_chunk(a + 1, chunk1, 1)
        serve(a + 1, chunk1)
        return carry

    lax.fori_loop(0, _ROUNDS // 2, rnd2, 0)
    wait_chunk(_ROUNDS - 1, chunk0, 0)
    serve(_ROUNDS - 1, chunk0)


def kernel(user_fea, embedding_userId):
    idx = user_fea[:, 0].astype(jnp.int32)
    tail = jnp.pad(embedding_userId[999936:].T, ((0, 0), (0, 64)))
    return _sc_gather(embedding_userId.T, tail, idx)
